# Initial kernel scaffold; baseline (speedup 1.0000x reference)
import jax
import jax.numpy as jnp
from jax.experimental import pallas as pl


def kernel(latents, W):
    raise NotImplementedError("write your pallas kernel here")


# speedup vs baseline: 1.2371x; 1.2371x over previous
import functools

import jax
import jax.numpy as jnp
from jax import lax
from jax.experimental import pallas as pl
from jax.experimental.pallas import tpu as pltpu
from jax.experimental.pallas import tpu_sc as plsc

_K = 8192
_D = 32
_N = 16384
_TN = 256
_G = _N // _TN
_NB = 2
_BK = _K // _NB
_COMMIT = 0.25

_NW = 32
_BPW = _N // _NW


def _dist_body(x_ref, x2_ref, w2x_ref, w2_ref, idx_ref, loss_ref, loss_acc):
    i = pl.program_id(0)
    x = x_ref[...]
    xb = x.astype(jnp.bfloat16).astype(jnp.float32)
    mm2 = jax.lax.dot_general(xb, w2x_ref[...], (((1,), (1,)), ((), ())),
                              preferred_element_type=jnp.float32)
    dmat = (x2_ref[...] + w2_ref[...]) - mm2

    rv = None
    rj = None
    wv = None
    for b in range(_NB):
        dblk = dmat[:, b * _BK:(b + 1) * _BK]
        bm = jnp.min(dblk, axis=1, keepdims=True)
        ii = jax.lax.broadcasted_iota(jnp.int32, (_TN, _BK), 1)
        bi = jnp.min(jnp.where(dblk == bm, ii, _BK), axis=1,
                     keepdims=True) + b * _BK
        if b == 0:
            rv, rj, wv = bm.astype(jnp.bfloat16), bi, bm
        else:
            pred = bm < rv.astype(jnp.float32)
            rv = jnp.where(pred, bm.astype(jnp.bfloat16), rv)
            rj = jnp.where(pred, bi, rj)
            wv = jnp.where(pred, bm, wv)
    idx_ref[...] = rj

    @pl.when(i == 0)
    def _init():
        loss_acc[0] = 0.0

    loss_acc[0] += jnp.sum(wv)

    @pl.when(i == _G - 1)
    def _finalize():
        loss_ref[0] = (1.0 + _COMMIT) * loss_acc[0] / (_N * _D)


def _sc_gather_hist(idx_hbm, w_hbm, zeros_hbm, ones_hbm, q_hbm, counts_hbm,
                    idx_v, rows_v, ones_v, shared, sem):
    cid = lax.axis_index("c")
    sid = lax.axis_index("s")
    wid = sid * 2 + cid
    base = wid * _BPW
    pltpu.sync_copy(idx_hbm.at[pl.ds(base, _BPW)], idx_v)
    pltpu.async_copy(w_hbm.at[idx_v], rows_v, sem).wait()
    pltpu.sync_copy(rows_v, q_hbm.at[pl.ds(base, _BPW)])

    @pl.when(sid == 0)
    def _init():
        pltpu.sync_copy(zeros_hbm, shared)

    plsc.subcore_barrier()
    pltpu.sync_copy(ones_hbm.at[pl.ds(base, _BPW)], ones_v)
    pltpu.sync_copy(ones_v, shared.at[idx_v], add=True)
    plsc.subcore_barrier()

    @pl.when(sid == 0)
    def _flush():
        pltpu.sync_copy(shared, counts_hbm.at[cid])


_NSC = 2


def _sc_call(idx_flat, W, zeros, ones):
    mesh = plsc.VectorSubcoreMesh(core_axis_name="c", subcore_axis_name="s")
    kern = functools.partial(
        pl.kernel,
        out_type=[
            jax.ShapeDtypeStruct((_N, 128), jnp.float32),
            jax.ShapeDtypeStruct((_NSC, _K), jnp.float32),
        ],
        mesh=mesh,
        scratch_types=[
            pltpu.VMEM((_BPW,), jnp.int32),
            pltpu.VMEM((_BPW, 128), jnp.float32),
            pltpu.VMEM((_BPW,), jnp.float32),
            pltpu.VMEM_SHARED((_K,), jnp.float32),
            pltpu.SemaphoreType.DMA,
        ],
    )(_sc_gather_hist)
    return kern(idx_flat, W, zeros, ones)


_TE = 4096
_GE = _N // _TE


def _epi_body(x_ref, q_ref, cp_ref, qst_ref, perp_ref):
    i = pl.program_id(0)
    x = x_ref[...]
    q = q_ref[..., :_D]
    qst_ref[...] = x + (q - x)

    @pl.when(i == 0)
    def _entropy():
        counts = jnp.sum(cp_ref[...], axis=0, keepdims=True)
        p = counts * (1.0 / _N)
        ent = -jnp.sum(p * jnp.log(p + 1e-10))
        perp_ref[0] = jnp.exp(ent)


def kernel(latents, W):
    B, C, H, Wd = latents.shape
    lat_t = jnp.transpose(latents, (0, 2, 3, 1))
    flat = lat_t.reshape(-1, C)
    x2 = jnp.sum(flat ** 2, axis=1, keepdims=True)
    w2 = jnp.sum(W ** 2, axis=1).reshape(1, -1)
    w2x = 2.0 * W

    idx2, loss = pl.pallas_call(
        _dist_body,
        grid=(_G,),
        in_specs=[
            pl.BlockSpec((_TN, _D), lambda i: (i, 0)),
            pl.BlockSpec((_TN, 1), lambda i: (i, 0)),
            pl.BlockSpec((_K, _D), lambda i: (0, 0)),
            pl.BlockSpec((1, _K), lambda i: (0, 0)),
        ],
        out_specs=[
            pl.BlockSpec((_TN, 1), lambda i: (i, 0)),
            pl.BlockSpec(memory_space=pltpu.SMEM),
        ],
        out_shape=[
            jax.ShapeDtypeStruct((_N, 1), jnp.int32),
            jax.ShapeDtypeStruct((1,), jnp.float32),
        ],
        scratch_shapes=[
            pltpu.SMEM((1,), jnp.float32),
        ],
    )(flat, x2, w2x, w2)

    zeros = jnp.zeros((_K,), jnp.float32)
    ones = jnp.ones((_N,), jnp.float32)
    w_pad = jnp.pad(W, ((0, 0), (0, 128 - _D)))
    q_pad, counts_p = _sc_call(idx2.reshape(-1), w_pad, zeros, ones)

    qst, perp = pl.pallas_call(
        _epi_body,
        grid=(_GE,),
        in_specs=[
            pl.BlockSpec((_TE, _D), lambda i: (i, 0)),
            pl.BlockSpec((_TE, 128), lambda i: (i, 0)),
            pl.BlockSpec((_NSC, _K), lambda i: (0, 0)),
        ],
        out_specs=[
            pl.BlockSpec((_TE, _D), lambda i: (i, 0)),
            pl.BlockSpec(memory_space=pltpu.SMEM),
        ],
        out_shape=[
            jax.ShapeDtypeStruct((_N, _D), jnp.float32),
            jax.ShapeDtypeStruct((1,), jnp.float32),
        ],
    )(flat, q_pad, counts_p)

    vq_loss = loss[0]
    perplexity = perp[0]
    idx_out = idx2.reshape(B, H, Wd)
    quantized_out = jnp.transpose(qst.reshape(B, H, Wd, C), (0, 3, 1, 2))
    return vq_loss, quantized_out, perplexity, idx_out


# speedup vs baseline: 1.2578x; 1.0167x over previous
import functools

import jax
import jax.numpy as jnp
from jax import lax
from jax.experimental import pallas as pl
from jax.experimental.pallas import tpu as pltpu
from jax.experimental.pallas import tpu_sc as plsc

_K = 8192
_D = 32
_N = 16384
_TN = 512
_G = _N // _TN
_NB = 2
_BK = _K // _NB
_COMMIT = 0.25

_NW = 32
_BPW = _N // _NW


def _dist_body(x_ref, x2_ref, w2x_ref, w2_ref, idx_ref, loss_ref, loss_acc):
    i = pl.program_id(0)
    x = x_ref[...]
    xb = x.astype(jnp.bfloat16).astype(jnp.float32)
    mm2 = jax.lax.dot_general(xb, w2x_ref[...], (((1,), (1,)), ((), ())),
                              preferred_element_type=jnp.float32)
    dmat = (x2_ref[...] + w2_ref[...]) - mm2

    rv = None
    rj = None
    wv = None
    for b in range(_NB):
        dblk = dmat[:, b * _BK:(b + 1) * _BK]
        bm = jnp.min(dblk, axis=1, keepdims=True)
        ii = jax.lax.broadcasted_iota(jnp.int32, (_TN, _BK), 1)
        bi = jnp.min(jnp.where(dblk == bm, ii, _BK), axis=1,
                     keepdims=True) + b * _BK
        if b == 0:
            rv, rj, wv = bm.astype(jnp.bfloat16), bi, bm
        else:
            pred = bm < rv.astype(jnp.float32)
            rv = jnp.where(pred, bm.astype(jnp.bfloat16), rv)
            rj = jnp.where(pred, bi, rj)
            wv = jnp.where(pred, bm, wv)
    idx_ref[...] = rj

    @pl.when(i == 0)
    def _init():
        loss_acc[0] = 0.0

    loss_acc[0] += jnp.sum(wv)

    @pl.when(i == _G - 1)
    def _finalize():
        loss_ref[0] = (1.0 + _COMMIT) * loss_acc[0] / (_N * _D)


def _sc_gather_hist(idx_hbm, w_hbm, zeros_hbm, ones_hbm, q_hbm, counts_hbm,
                    idx_v, rows_v, ones_v, shared, sem):
    cid = lax.axis_index("c")
    sid = lax.axis_index("s")
    wid = sid * 2 + cid
    base = wid * _BPW
    pltpu.sync_copy(idx_hbm.at[pl.ds(base, _BPW)], idx_v)
    pltpu.async_copy(w_hbm.at[idx_v], rows_v, sem).wait()
    pltpu.sync_copy(rows_v, q_hbm.at[pl.ds(base, _BPW)])

    @pl.when(sid == 0)
    def _init():
        pltpu.sync_copy(zeros_hbm, shared)

    plsc.subcore_barrier()
    pltpu.sync_copy(ones_hbm.at[pl.ds(base, _BPW)], ones_v)
    pltpu.sync_copy(ones_v, shared.at[idx_v], add=True)
    plsc.subcore_barrier()

    @pl.when(sid == 0)
    def _flush():
        pltpu.sync_copy(shared, counts_hbm.at[cid])


_NSC = 2


def _sc_call(idx_flat, W, zeros, ones):
    mesh = plsc.VectorSubcoreMesh(core_axis_name="c", subcore_axis_name="s")
    kern = functools.partial(
        pl.kernel,
        out_type=[
            jax.ShapeDtypeStruct((_N, 128), jnp.float32),
            jax.ShapeDtypeStruct((_NSC, _K), jnp.float32),
        ],
        mesh=mesh,
        scratch_types=[
            pltpu.VMEM((_BPW,), jnp.int32),
            pltpu.VMEM((_BPW, 128), jnp.float32),
            pltpu.VMEM((_BPW,), jnp.float32),
            pltpu.VMEM_SHARED((_K,), jnp.float32),
            pltpu.SemaphoreType.DMA,
        ],
    )(_sc_gather_hist)
    return kern(idx_flat, W, zeros, ones)


_TE = 4096
_GE = _N // _TE


def _epi_body(x_ref, q_ref, cp_ref, qst_ref, perp_ref):
    i = pl.program_id(0)
    x = x_ref[...]
    q = q_ref[..., :_D]
    qst_ref[...] = x + (q - x)

    @pl.when(i == 0)
    def _entropy():
        counts = jnp.sum(cp_ref[...], axis=0, keepdims=True)
        p = counts * (1.0 / _N)
        ent = -jnp.sum(p * jnp.log(p + 1e-10))
        perp_ref[0] = jnp.exp(ent)


def kernel(latents, W):
    B, C, H, Wd = latents.shape
    lat_t = jnp.transpose(latents, (0, 2, 3, 1))
    flat = lat_t.reshape(-1, C)
    x2 = jnp.sum(flat ** 2, axis=1, keepdims=True)
    w2 = jnp.sum(W ** 2, axis=1).reshape(1, -1)
    w2x = 2.0 * W

    idx2, loss = pl.pallas_call(
        _dist_body,
        grid=(_G,),
        in_specs=[
            pl.BlockSpec((_TN, _D), lambda i: (i, 0)),
            pl.BlockSpec((_TN, 1), lambda i: (i, 0)),
            pl.BlockSpec((_K, _D), lambda i: (0, 0)),
            pl.BlockSpec((1, _K), lambda i: (0, 0)),
        ],
        out_specs=[
            pl.BlockSpec((_TN, 1), lambda i: (i, 0)),
            pl.BlockSpec(memory_space=pltpu.SMEM),
        ],
        out_shape=[
            jax.ShapeDtypeStruct((_N, 1), jnp.int32),
            jax.ShapeDtypeStruct((1,), jnp.float32),
        ],
        scratch_shapes=[
            pltpu.SMEM((1,), jnp.float32),
        ],
    )(flat, x2, w2x, w2)

    zeros = jnp.zeros((_K,), jnp.float32)
    ones = jnp.ones((_N,), jnp.float32)
    w_pad = jnp.pad(W, ((0, 0), (0, 128 - _D)))
    q_pad, counts_p = _sc_call(idx2.reshape(-1), w_pad, zeros, ones)

    qst, perp = pl.pallas_call(
        _epi_body,
        grid=(_GE,),
        in_specs=[
            pl.BlockSpec((_TE, _D), lambda i: (i, 0)),
            pl.BlockSpec((_TE, 128), lambda i: (i, 0)),
            pl.BlockSpec((_NSC, _K), lambda i: (0, 0)),
        ],
        out_specs=[
            pl.BlockSpec((_TE, _D), lambda i: (i, 0)),
            pl.BlockSpec(memory_space=pltpu.SMEM),
        ],
        out_shape=[
            jax.ShapeDtypeStruct((_N, _D), jnp.float32),
            jax.ShapeDtypeStruct((1,), jnp.float32),
        ],
    )(flat, q_pad, counts_p)

    vq_loss = loss[0]
    perplexity = perp[0]
    idx_out = idx2.reshape(B, H, Wd)
    quantized_out = jnp.transpose(qst.reshape(B, H, Wd, C), (0, 3, 1, 2))
    return vq_loss, quantized_out, perplexity, idx_out
